# baseline (speedup 1.0000x reference)
import jax
import jax.numpy as jnp
from jax.experimental import pallas as pl

_D = _H = _W = 128
_C = 12
_N = 2097152
_P = 256
_XS = 16
_NSLAB = _D // _XS
_NCHUNK = _N // _P
_ROWS = (_C + 1) * _XS


def _splat_kernel(coords_ref, vals_ref, out_ref):
    s = pl.program_id(0)
    k = pl.program_id(1)

    coords = coords_ref[...]
    vals = vals_ref[...]

    sizes = jnp.float32(_D)
    ind = (coords - (-1.0)) / 2.0 * (sizes - 1.0)
    ind = jnp.clip(ind, 0.0, sizes - 1.0 - 1e-4)
    i0f = jnp.floor(ind)
    frac = ind - i0f
    i0 = i0f.astype(jnp.int32)

    xcols = s * _XS + jax.lax.broadcasted_iota(jnp.int32, (_P, _XS), 1)
    i0x = i0[:, 0:1]
    fx = frac[:, 0:1]
    ax = jnp.where(xcols == i0x, 1.0 - fx, 0.0) + jnp.where(
        xcols == i0x + 1, fx, 0.0)

    ycols = jax.lax.broadcasted_iota(jnp.int32, (_P, _H), 1)
    i0y = i0[:, 1:2]
    fy = frac[:, 1:2]
    ay = jnp.where(ycols == i0y, 1.0 - fy, 0.0) + jnp.where(
        ycols == i0y + 1, fy, 0.0)
    zcols = jax.lax.broadcasted_iota(jnp.int32, (_P, _W), 1)
    i0z = i0[:, 2:3]
    fz = frac[:, 2:3]
    az = jnp.where(zcols == i0z, 1.0 - fz, 0.0) + jnp.where(
        zcols == i0z + 1, fz, 0.0)

    wyz = (ay[:, :, None] * az[:, None, :]).reshape(_P, _H * _W)

    v13 = jnp.concatenate([vals, jnp.ones((_P, 1), jnp.float32)], axis=1)
    a = (v13[:, :, None] * ax[:, None, :]).reshape(_P, _ROWS)

    contrib = jax.lax.dot_general(
        a.astype(jnp.bfloat16), wyz.astype(jnp.bfloat16),
        dimension_numbers=(((0,), (0,)), ((), ())),
        preferred_element_type=jnp.float32)

    @pl.when(k == 0)
    def _init():
        out_ref[0] = contrib

    @pl.when(k > 0)
    def _acc():
        out_ref[0] = out_ref[0] + contrib

    @pl.when(k == _NCHUNK - 1)
    def _normalize():
        acc = out_ref[0]
        wsum = acc[_C * _XS:, :]
        wrep = jnp.concatenate([wsum] * _C, axis=0)
        out_ref[0, 0:_C * _XS, :] = acc[0:_C * _XS, :] / (wrep + 1e-8)


def kernel(particle_coords, particle_values):
    out = pl.pallas_call(
        _splat_kernel,
        grid=(_NSLAB, _NCHUNK),
        in_specs=[
            pl.BlockSpec((_P, 3), lambda s, k: (k, 0)),
            pl.BlockSpec((_P, _C), lambda s, k: (k, 0)),
        ],
        out_specs=pl.BlockSpec((1, _ROWS, _H * _W), lambda s, k: (s, 0, 0)),
        out_shape=jax.ShapeDtypeStruct((_NSLAB, _ROWS, _H * _W), jnp.float32),
    )(particle_coords, particle_values)

    grid = out[:, :_C * _XS, :].reshape(_NSLAB, _C, _XS, _H * _W)
    grid = jnp.transpose(grid, (1, 0, 2, 3)).reshape(_C, _D, _H, _W)
    return grid[None]


# speedup vs baseline: 1.8425x; 1.8425x over previous
import jax
import jax.numpy as jnp
from jax.experimental import pallas as pl

_D = _H = _W = 128
_C = 12
_N = 2097152
_P = 1024
_XS = 16
_NSLAB = _D // _XS
_NCHUNK = _N // _P
_ROWS = (_C + 1) * _XS


def _splat_kernel(coords_ref, vals_ref, out_ref):
    s = pl.program_id(0)
    k = pl.program_id(1)

    coords = coords_ref[...]
    vals = vals_ref[...]

    sizes = jnp.float32(_D)
    ind = (coords - (-1.0)) / 2.0 * (sizes - 1.0)
    ind = jnp.clip(ind, 0.0, sizes - 1.0 - 1e-4)
    i0f = jnp.floor(ind)
    frac = ind - i0f
    i0 = i0f.astype(jnp.int32)

    xcols = s * _XS + jax.lax.broadcasted_iota(jnp.int32, (_P, _XS), 1)
    i0x = i0[:, 0:1]
    fx = frac[:, 0:1]
    ax = jnp.where(xcols == i0x, 1.0 - fx, 0.0) + jnp.where(
        xcols == i0x + 1, fx, 0.0)

    ycols = jax.lax.broadcasted_iota(jnp.int32, (_P, _H), 1)
    i0y = i0[:, 1:2]
    fy = frac[:, 1:2]
    ay = jnp.where(ycols == i0y, 1.0 - fy, 0.0) + jnp.where(
        ycols == i0y + 1, fy, 0.0)
    zcols = jax.lax.broadcasted_iota(jnp.int32, (_P, _W), 1)
    i0z = i0[:, 2:3]
    fz = frac[:, 2:3]
    az = jnp.where(zcols == i0z, 1.0 - fz, 0.0) + jnp.where(
        zcols == i0z + 1, fz, 0.0)

    ay16 = ay.astype(jnp.bfloat16)
    az16 = az.astype(jnp.bfloat16)
    wyz = (ay16[:, :, None] * az16[:, None, :]).reshape(_P, _H * _W)

    v13 = jnp.concatenate([vals, jnp.ones((_P, 1), jnp.float32)], axis=1)
    a = (v13.astype(jnp.bfloat16)[:, :, None] *
         ax.astype(jnp.bfloat16)[:, None, :]).reshape(_P, _ROWS)

    contrib = jax.lax.dot_general(
        a, wyz, dimension_numbers=(((0,), (0,)), ((), ())),
        preferred_element_type=jnp.float32)

    @pl.when(k == 0)
    def _init():
        out_ref[0] = contrib

    @pl.when(k > 0)
    def _acc():
        out_ref[0] = out_ref[0] + contrib

    @pl.when(k == _NCHUNK - 1)
    def _normalize():
        acc = out_ref[0]
        wsum = acc[_C * _XS:, :]
        wrep = jnp.concatenate([wsum] * _C, axis=0)
        out_ref[0, 0:_C * _XS, :] = acc[0:_C * _XS, :] / (wrep + 1e-8)


def kernel(particle_coords, particle_values):
    out = pl.pallas_call(
        _splat_kernel,
        grid=(_NSLAB, _NCHUNK),
        in_specs=[
            pl.BlockSpec((_P, 3), lambda s, k: (k, 0)),
            pl.BlockSpec((_P, _C), lambda s, k: (k, 0)),
        ],
        out_specs=pl.BlockSpec((1, _ROWS, _H * _W), lambda s, k: (s, 0, 0)),
        out_shape=jax.ShapeDtypeStruct((_NSLAB, _ROWS, _H * _W), jnp.float32),
    )(particle_coords, particle_values)

    grid = out[:, :_C * _XS, :].reshape(_NSLAB, _C, _XS, _H * _W)
    grid = jnp.transpose(grid, (1, 0, 2, 3)).reshape(_C, _D, _H, _W)
    return grid[None]
